# Initial kernel scaffold; baseline (speedup 1.0000x reference)
import jax
import jax.numpy as jnp
from jax.experimental import pallas as pl


def kernel(x, mask, points, conv_w, conv_b, plin_w, plin_b, n1_s, n1_b, qkv_w, qkv_b, ap_w, ap_b, n2_s, n2_b, fc1_w, fc1_b, fc2_w, fc2_b, proj_mask, proj_points, dn_s, dn_b, mn_s, mn_b, mlp_w, mlp_b):
    raise NotImplementedError("write your pallas kernel here")


# speedup vs baseline: 1.1659x; 1.1659x over previous
import math

import jax
import jax.numpy as jnp
from jax.experimental import pallas as pl
from jax.experimental.pallas import tpu as pltpu

B = 32; IMG = 224; PATCH = 16; D = 48; HEADS = 12; MLPD = 192; L = 6
NPTS = 48; NPATCH = (IMG // PATCH) ** 2
VAR_BIAS = 4.0; EPS = 1e-5; HALF = NPTS // 2


def _ln(x, s, b):
    m = x.mean(-1, keepdims=True)
    v = ((x - m) ** 2).mean(-1, keepdims=True)
    return (x - m) / jnp.sqrt(v + EPS) * s + b


def _gauss_kernel(pts_ref, var_ref, out_ref):
    pts = pts_ref[0]
    var_row = var_ref[0]
    ii48 = jax.lax.broadcasted_iota(jnp.int32, (NPTS, NPTS), 0)
    jj48 = jax.lax.broadcasted_iota(jnp.int32, (NPTS, NPTS), 1)
    eye48 = jnp.where(ii48 == jj48, 1.0, 0.0)
    var_col = jax.lax.dot_general(eye48, var_row, (((1,), (1,)), ((), ())),
                                  preferred_element_type=jnp.float32)

    ii = jax.lax.broadcasted_iota(jnp.int32, (HALF, HALF), 0)
    jj = jax.lax.broadcasted_iota(jnp.int32, (HALF, HALF), 1)
    lower_tri = jnp.where(jj <= ii, 1.0, 0.0)
    col_iota = jax.lax.broadcasted_iota(jnp.int32, (HALF, IMG), 1).astype(jnp.float32)

    for phase in range(2):
        p0 = phase * HALF
        pr = pts[p0:p0 + HALF, 0:1]
        pc = pts[p0:p0 + HALF, 1:2]
        vcol = var_col[p0:p0 + HALF]
        valid = jnp.where(jnp.maximum(pr, pc) > 0, 1.0, 0.0)
        cum = jax.lax.dot_general(lower_tri, valid, (((1,), (0,)), ((), ())),
                                  preferred_element_type=jnp.float32)
        rank = jnp.clip(cum - 1.0, 0.0, HALF - 1.0)
        onehot = jnp.where(jj == rank.astype(jnp.int32), 1.0, 0.0)
        var_p = jax.lax.dot_general(onehot, vcol, (((1,), (0,)), ((), ())),
                                    preferred_element_type=jnp.float32) + VAR_BIAS
        v2 = 2.0 * var_p * var_p
        nv = jnp.sum(valid)
        scale = valid / (math.pi * v2 * jnp.maximum(nv, 1.0))
        ar = jnp.exp(-((col_iota - pr) ** 2) / v2) * scale
        ac = jnp.exp(-((col_iota - pc) ** 2) / v2)
        h = jax.lax.dot_general(ar, ac, (((0,), (0,)), ((), ())),
                                preferred_element_type=jnp.float32)
        mn = jnp.min(h)
        mx = jnp.max(h)
        denom = jnp.where(mx > mn, mx - mn, 1.0)
        res = 2.0 * (h - mn) / denom - 1.0
        res = jnp.where(nv > 0, res, 0.0)
        out_ref[0, phase] = res


def kernel(x, mask, points, conv_w, conv_b, plin_w, plin_b, n1_s, n1_b,
           qkv_w, qkv_b, ap_w, ap_b, n2_s, n2_b, fc1_w, fc1_b, fc2_w, fc2_b,
           proj_mask, proj_points, dn_s, dn_b, mn_s, mn_b, mlp_w, mlp_b):
    Bsz = x.shape[0]
    mp = jax.lax.conv_general_dilated(mask, conv_w, (PATCH, PATCH), 'VALID',
                                      dimension_numbers=('NCHW', 'OIHW', 'NCHW'))
    mp = mp + conv_b[None, :, None, None]
    mask_embed = mp.reshape(Bsz, D, -1).transpose(0, 2, 1)
    points_embed = points @ plin_w + plin_b
    feat = jnp.concatenate([mask_embed, points_embed], axis=1)
    hd = D // HEADS
    sc = hd ** -0.5
    for i in range(L):
        h = _ln(feat, n1_s[i], n1_b[i])
        qkv = (h @ qkv_w[i] + qkv_b[i]).reshape(Bsz, -1, 3, HEADS, hd)
        q = qkv[:, :, 0].transpose(0, 2, 1, 3)
        k = qkv[:, :, 1].transpose(0, 2, 1, 3)
        v = qkv[:, :, 2].transpose(0, 2, 1, 3)
        a = jax.nn.softmax(jnp.einsum('bhnd,bhmd->bhnm', q, k) * sc, axis=-1)
        y = jnp.einsum('bhnm,bhmd->bhnd', a, v).transpose(0, 2, 1, 3).reshape(Bsz, -1, D)
        feat = feat + (y @ ap_w[i] + ap_b[i])
        h = _ln(feat, n2_s[i], n2_b[i])
        feat = feat + (jax.nn.gelu(h @ fc1_w[i] + fc1_b[i], approximate=False) @ fc2_w[i] + fc2_b[i])
    feat = _ln(feat, dn_s, dn_b)
    mf, pf = feat[:, :NPATCH], feat[:, NPATCH:]
    mf = mf @ proj_mask
    pf = pf @ proj_points
    mf = mf / jnp.linalg.norm(mf, axis=-1, keepdims=True)
    pf = pf / jnp.linalg.norm(pf, axis=-1, keepdims=True)
    point_mask = _ln(jnp.einsum('bnd,bpd->bnp', mf, pf), mn_s, mn_b)
    vars_ = jnp.clip(jax.nn.gelu(point_mask.reshape(Bsz, -1), approximate=False) @ mlp_w + mlp_b, 0.0, 4.0)

    vars3 = vars_.reshape(Bsz, 1, NPTS)
    out = pl.pallas_call(
        _gauss_kernel,
        grid=(Bsz,),
        in_specs=[pl.BlockSpec((1, NPTS, 3), lambda b: (b, 0, 0)),
                  pl.BlockSpec((1, 1, NPTS), lambda b: (b, 0, 0))],
        out_specs=pl.BlockSpec((1, 2, IMG, IMG), lambda b: (b, 0, 0, 0)),
        out_shape=jax.ShapeDtypeStruct((Bsz, 2, IMG, IMG), jnp.float32),
        compiler_params=pltpu.CompilerParams(dimension_semantics=('arbitrary',)),
    )(points, vars3)
    return out


# speedup vs baseline: 1.5908x; 1.3644x over previous
import math

import jax
import jax.numpy as jnp
from jax.experimental import pallas as pl
from jax.experimental.pallas import tpu as pltpu

IMG = 224; PATCH = 16; D = 48; HEADS = 12; MLPD = 192; L = 6
NPTS = 48; NPATCH = (IMG // PATCH) ** 2
VAR_BIAS = 4.0; EPS = 1e-5; HALF = NPTS // 2
SEQ = NPATCH + NPTS
SEQP = 256
HD = D // HEADS
BB = 1
LOG2E = 1.4426950408889634


def _erf(x):
    a = jnp.abs(x)
    t = 1.0 / (1.0 + 0.3275911 * a)
    poly = ((((1.061405429 * t - 1.453152027) * t + 1.421413741) * t
             - 0.284496736) * t + 0.254829592) * t
    y = 1.0 - poly * jnp.exp(-a * a)
    return jnp.where(x < 0, -y, y)


def _gelu(x):
    return 0.5 * x * (1.0 + _erf(x * (1.0 / math.sqrt(2.0))))


def _dot(a, b):
    return jax.lax.dot_general(a, b, (((1,), (0,)), ((), ())),
                               preferred_element_type=jnp.float32)


def _dot_tn(a, b):
    return jax.lax.dot_general(a, b, (((0,), (0,)), ((), ())),
                               preferred_element_type=jnp.float32)


def _dot_nt(a, b):
    return jax.lax.dot_general(a, b, (((1,), (1,)), ((), ())),
                               preferred_element_type=jnp.float32)


def _fused_kernel(patches_ref, pts_ref, conv_wf_ref, conv_b_ref,
                  plin_w_ref, plin_b_ref, n1_s_ref, n1_b_ref,
                  qkv_w_ref, qkv_b_ref, ap_w_ref, ap_b_ref,
                  n2_s_ref, n2_b_ref, fc1_w_ref, fc1_b_ref,
                  fc2_w_ref, fc2_b_ref, proj_m_ref, proj_p_ref,
                  dn_s_ref, dn_b_ref, mn_s_ref, mn_b_ref,
                  w3f_ref, mlp_b_ref, out_ref):
    f32 = jnp.float32
    avg48 = jnp.full((D, D), 1.0 / D, f32)
    ones48 = jnp.full((D, D), 1.0, f32)
    ones_col = jnp.full((SEQP, HD), 1.0, f32)
    row2 = jax.lax.broadcasted_iota(jnp.int32, (SEQP, D), 0)
    rowmask = row2 < NPATCH
    colj = jax.lax.broadcasted_iota(jnp.int32, (SEQP, NPTS), 1)
    selT = jnp.where(row2[:, :NPTS] == colj + NPATCH, 1.0, 0.0)
    kcol = jax.lax.broadcasted_iota(jnp.int32, (SEQP, SEQP), 1)
    kmask = kcol < SEQ
    ii_a = jax.lax.broadcasted_iota(jnp.int32, (NPTS, NPTS * NPTS), 0)
    jj_a = jax.lax.broadcasted_iota(jnp.int32, (NPTS, NPTS * NPTS), 1)
    amask = jnp.where(jj_a // NPTS == ii_a, 1.0, 0.0)
    qq = jax.lax.broadcasted_iota(jnp.int32, (NPTS * NPTS, NPTS), 0)
    jj_b = jax.lax.broadcasted_iota(jnp.int32, (NPTS * NPTS, NPTS), 1)
    bmask = jnp.where(qq % NPTS == jj_b, 1.0, 0.0)
    ii48 = jax.lax.broadcasted_iota(jnp.int32, (NPTS, NPTS), 0)
    jj48 = jax.lax.broadcasted_iota(jnp.int32, (NPTS, NPTS), 1)
    eye48 = jnp.where(ii48 == jj48, 1.0, 0.0)
    ii = ii48[:HALF, :HALF]
    jj = jj48[:HALF, :HALF]
    lower_tri = jnp.where(jj <= ii, 1.0, 0.0)
    col_iota = jax.lax.broadcasted_iota(jnp.int32, (HALF, IMG), 1).astype(f32)
    scale = HD ** -0.5

    def _ln(x, s, b):
        m = _dot(x, avg48)
        d = x - m
        v = _dot(d * d, avg48)
        return d * jax.lax.rsqrt(v + EPS) * s + b

    for g in range(BB):
        femb = _dot(patches_ref[g], conv_wf_ref[...]) + conv_b_ref[...]
        pemb = _dot(pts_ref[g], plin_w_ref[...]) + plin_b_ref[...]
        feat = jnp.where(rowmask, femb, 0.0) + _dot(selT, pemb)

        for i in range(L):
            h = _ln(feat, n1_s_ref[i], n1_b_ref[i])
            qkv = _dot(h, qkv_w_ref[i]) + qkv_b_ref[i]
            ys = []
            for hd in range(HEADS):
                qh = qkv[:, HD * hd:HD * hd + HD]
                kh = qkv[:, D + HD * hd:D + HD * hd + HD]
                vh = qkv[:, 2 * D + HD * hd:2 * D + HD * hd + HD]
                s = _dot_nt(qh, kh)
                e = jnp.where(kmask, jnp.exp2(s * (scale * LOG2E)), 0.0)
                vh8 = jnp.concatenate([vh, ones_col], axis=1)
                o8 = _dot(e, vh8)
                ys.append(o8[:, :HD] / o8[:, HD:])
            y = jnp.concatenate(ys, axis=1)
            feat = feat + _dot(y, ap_w_ref[i]) + ap_b_ref[i]
            h2 = _ln(feat, n2_s_ref[i], n2_b_ref[i])
            gm = _gelu(_dot(h2, fc1_w_ref[i]) + fc1_b_ref[i])
            feat = feat + _dot(gm, fc2_w_ref[i]) + fc2_b_ref[i]

        feat = _ln(feat, dn_s_ref[...], dn_b_ref[...])
        mfull = _dot(feat, proj_m_ref[...])
        pfull = _dot(feat, proj_p_ref[...])
        mfn = mfull * jax.lax.rsqrt(_dot(mfull * mfull, ones48))
        pfn = pfull * jax.lax.rsqrt(_dot(pfull * pfull, ones48))
        pf = _dot_tn(selT, pfn)
        pm = _dot_nt(mfn, pf)
        pm = _ln(pm, mn_s_ref[...], mn_b_ref[...])
        gpm = jnp.where(rowmask, _gelu(pm), 0.0)

        r = _dot_tn(gpm, w3f_ref[...])
        t = jnp.sum(r * amask, axis=0, keepdims=True)
        vars_row = jnp.clip(_dot(t, bmask) + mlp_b_ref[...], 0.0, 4.0)

        pts = pts_ref[g]
        var_col = _dot_nt(eye48, vars_row)

        for phase in range(2):
            p0 = phase * HALF
            pr = pts[p0:p0 + HALF, 0:1]
            pc = pts[p0:p0 + HALF, 1:2]
            vcol = var_col[p0:p0 + HALF]
            valid = jnp.where(jnp.maximum(pr, pc) > 0, 1.0, 0.0)
            cum = _dot(lower_tri, valid)
            rank = jnp.clip(cum - 1.0, 0.0, HALF - 1.0)
            onehot = jnp.where(jj == rank.astype(jnp.int32), 1.0, 0.0)
            var_p = _dot(onehot, vcol) + VAR_BIAS
            v2 = 2.0 * var_p * var_p
            nv = jnp.sum(valid)
            sc = valid / (math.pi * v2 * jnp.maximum(nv, 1.0))
            ar = jnp.exp(-((col_iota - pr) ** 2) / v2) * sc
            ac = jnp.exp(-((col_iota - pc) ** 2) / v2)
            hmap = _dot_tn(ar, ac)
            mn = jnp.min(hmap)
            mx = jnp.max(hmap)
            denom = jnp.where(mx > mn, mx - mn, 1.0)
            res = 2.0 * (hmap - mn) / denom - 1.0
            out_ref[g, phase] = jnp.where(nv > 0, res, 0.0)


def kernel(x, mask, points, conv_w, conv_b, plin_w, plin_b, n1_s, n1_b,
           qkv_w, qkv_b, ap_w, ap_b, n2_s, n2_b, fc1_w, fc1_b, fc2_w, fc2_b,
           proj_mask, proj_points, dn_s, dn_b, mn_s, mn_b, mlp_w, mlp_b):
    Bsz = x.shape[0]
    nh = IMG // PATCH
    patches = mask.reshape(Bsz, nh, PATCH, nh, PATCH).transpose(0, 1, 3, 2, 4)
    patches = patches.reshape(Bsz, NPATCH, PATCH * PATCH)
    patches = jnp.pad(patches, ((0, 0), (0, SEQP - NPATCH), (0, 0)))
    conv_wf = conv_w.reshape(D, PATCH * PATCH).T
    w3f = mlp_w.reshape(NPATCH, NPTS, NPTS).reshape(NPATCH, NPTS * NPTS)
    w3f = jnp.pad(w3f, ((0, SEQP - NPATCH), (0, 0)))

    full = lambda a: pl.BlockSpec(a.shape, lambda b: (0,) * a.ndim)
    args = [
        (patches, pl.BlockSpec((BB, SEQP, PATCH * PATCH), lambda b: (b, 0, 0))),
        (points, pl.BlockSpec((BB, NPTS, 3), lambda b: (b, 0, 0))),
        (conv_wf, full(conv_wf)),
        (conv_b.reshape(1, D), None),
        (plin_w, full(plin_w)),
        (plin_b.reshape(1, D), None),
        (n1_s.reshape(L, 1, D), None),
        (n1_b.reshape(L, 1, D), None),
        (qkv_w, full(qkv_w)),
        (qkv_b.reshape(L, 1, 3 * D), None),
        (ap_w, full(ap_w)),
        (ap_b.reshape(L, 1, D), None),
        (n2_s.reshape(L, 1, D), None),
        (n2_b.reshape(L, 1, D), None),
        (fc1_w, full(fc1_w)),
        (fc1_b.reshape(L, 1, MLPD), None),
        (fc2_w, full(fc2_w)),
        (fc2_b.reshape(L, 1, D), None),
        (proj_mask, full(proj_mask)),
        (proj_points, full(proj_points)),
        (dn_s.reshape(1, D), None),
        (dn_b.reshape(1, D), None),
        (mn_s.reshape(1, NPTS), None),
        (mn_b.reshape(1, NPTS), None),
        (w3f, full(w3f)),
        (mlp_b.reshape(1, NPTS), None),
    ]
    ins = [a for a, _ in args]
    specs = [s if s is not None else full(a) for a, s in args]
    out = pl.pallas_call(
        _fused_kernel,
        grid=(Bsz // BB,),
        in_specs=specs,
        out_specs=pl.BlockSpec((BB, 2, IMG, IMG), lambda b: (b, 0, 0, 0)),
        out_shape=jax.ShapeDtypeStruct((Bsz, 2, IMG, IMG), jnp.float32),
        compiler_params=pltpu.CompilerParams(
            dimension_semantics=('arbitrary',),
            vmem_limit_bytes=56 * 1024 * 1024,
        ),
    )(*ins)
    return out


# speedup vs baseline: 1.7322x; 1.0889x over previous
import math

import jax
import jax.numpy as jnp
from jax.experimental import pallas as pl
from jax.experimental.pallas import tpu as pltpu

IMG = 224; PATCH = 16; D = 48; HEADS = 12; MLPD = 192; L = 6
NPTS = 48; NPATCH = (IMG // PATCH) ** 2
VAR_BIAS = 4.0; EPS = 1e-5; HALF = NPTS // 2
SEQ = NPATCH + NPTS
SEQP = 256
HD = D // HEADS
BB = 1
LOG2E = 1.4426950408889634


def _erf(x):
    a = jnp.abs(x)
    t = 1.0 / (1.0 + 0.3275911 * a)
    poly = ((((1.061405429 * t - 1.453152027) * t + 1.421413741) * t
             - 0.284496736) * t + 0.254829592) * t
    y = 1.0 - poly * jnp.exp(-a * a)
    return jnp.where(x < 0, -y, y)


def _gelu(x):
    return 0.5 * x * (1.0 + _erf(x * (1.0 / math.sqrt(2.0))))


def _dot(a, b):
    return jax.lax.dot_general(a, b, (((1,), (0,)), ((), ())),
                               preferred_element_type=jnp.float32)


def _dot_tn(a, b):
    return jax.lax.dot_general(a, b, (((0,), (0,)), ((), ())),
                               preferred_element_type=jnp.float32)


def _dot_nt(a, b):
    return jax.lax.dot_general(a, b, (((1,), (1,)), ((), ())),
                               preferred_element_type=jnp.float32)


def _dotb_tn(a, b):
    return jax.lax.dot_general(a.astype(jnp.bfloat16), b.astype(jnp.bfloat16),
                               (((0,), (0,)), ((), ())),
                               preferred_element_type=jnp.float32)


def _dotb(a, b):
    return jax.lax.dot_general(a.astype(jnp.bfloat16), b.astype(jnp.bfloat16),
                               (((1,), (0,)), ((), ())),
                               preferred_element_type=jnp.float32)


def _dotb_nt(a, b):
    return jax.lax.dot_general(a.astype(jnp.bfloat16), b.astype(jnp.bfloat16),
                               (((1,), (1,)), ((), ())),
                               preferred_element_type=jnp.float32)


def _fused_kernel(patches_ref, pts_ref, conv_wf_ref, conv_b_ref,
                  plin_w_ref, plin_b_ref, n1_s_ref, n1_b_ref,
                  qkv_w_ref, qkv_b_ref, ap_w_ref, ap_b_ref,
                  n2_s_ref, n2_b_ref, fc1_w_ref, fc1_b_ref,
                  fc2_w_ref, fc2_b_ref, proj_m_ref, proj_p_ref,
                  dn_s_ref, dn_b_ref, mn_s_ref, mn_b_ref,
                  w3f_ref, mlp_b_ref, out_ref):
    f32 = jnp.float32
    avg48 = jnp.full((D, D), 1.0 / D, f32)
    ones48 = jnp.full((D, D), 1.0, f32)
    ones_blk = jnp.full((SEQP, D), 1.0, f32)
    hrow = jax.lax.broadcasted_iota(jnp.int32, (HEADS * SEQP, D), 0) // SEQP
    hcol = jax.lax.broadcasted_iota(jnp.int32, (HEADS * SEQP, D), 1) // HD
    hmask = hrow == hcol
    hcol3 = jax.lax.broadcasted_iota(jnp.int32, (HEADS, 1, D), 2) // HD
    hidx3 = jax.lax.broadcasted_iota(jnp.int32, (HEADS, 1, D), 0)
    hmask3 = hidx3 == hcol3
    kcol_all = jax.lax.broadcasted_iota(jnp.int32, (HEADS * SEQP, SEQP), 1)
    kmask_all = kcol_all < SEQ
    row2 = jax.lax.broadcasted_iota(jnp.int32, (SEQP, D), 0)
    rowmask = row2 < NPATCH
    colj = jax.lax.broadcasted_iota(jnp.int32, (SEQP, NPTS), 1)
    selT = jnp.where(row2[:, :NPTS] == colj + NPATCH, 1.0, 0.0)
    kcol = jax.lax.broadcasted_iota(jnp.int32, (SEQP, SEQP), 1)
    kmask = kcol < SEQ
    ii_a = jax.lax.broadcasted_iota(jnp.int32, (NPTS, NPTS * NPTS), 0)
    jj_a = jax.lax.broadcasted_iota(jnp.int32, (NPTS, NPTS * NPTS), 1)
    amask = jnp.where(jj_a // NPTS == ii_a, 1.0, 0.0)
    qq = jax.lax.broadcasted_iota(jnp.int32, (NPTS * NPTS, NPTS), 0)
    jj_b = jax.lax.broadcasted_iota(jnp.int32, (NPTS * NPTS, NPTS), 1)
    bmask = jnp.where(qq % NPTS == jj_b, 1.0, 0.0)
    ii48 = jax.lax.broadcasted_iota(jnp.int32, (NPTS, NPTS), 0)
    jj48 = jax.lax.broadcasted_iota(jnp.int32, (NPTS, NPTS), 1)
    eye48 = jnp.where(ii48 == jj48, 1.0, 0.0)
    ii = ii48[:HALF, :HALF]
    jj = jj48[:HALF, :HALF]
    lower_tri = jnp.where(jj <= ii, 1.0, 0.0)
    col_iota = jax.lax.broadcasted_iota(jnp.int32, (HALF, IMG), 1).astype(f32)
    scale = HD ** -0.5

    def _ln(x, s, b):
        m = _dotb(x, avg48)
        d = x - m
        v = _dotb(d * d, avg48)
        return d * jax.lax.rsqrt(v + EPS) * s + b

    for g in range(BB):
        femb = _dotb(patches_ref[g], conv_wf_ref[...]) + conv_b_ref[...]
        pemb = _dot(pts_ref[g], plin_w_ref[...]) + plin_b_ref[...]
        feat = jnp.where(rowmask, femb, 0.0) + _dot(selT, pemb)

        for i in range(L):
            h = _ln(feat, n1_s_ref[i], n1_b_ref[i])
            qkv = _dotb(h, qkv_w_ref[i]) + qkv_b_ref[i]
            q = qkv[:, :D]
            k = qkv[:, D:2 * D]
            v = qkv[:, 2 * D:3 * D]
            qbd = jnp.where(hmask, jnp.concatenate([q] * HEADS, axis=0), 0.0)
            s_all = _dotb_nt(qbd, k)
            e = jnp.where(kmask_all, jnp.exp2(s_all * (scale * LOG2E)), 0.0)
            rhs = jnp.concatenate([v, ones_blk], axis=1)
            o = _dotb(e, rhs)
            yn = o[:, :D] / o[:, D:2 * D]
            y3 = yn.reshape(HEADS, SEQP, D)
            y = jnp.sum(jnp.where(hmask3, y3, 0.0), axis=0)
            feat = feat + _dotb(y, ap_w_ref[i]) + ap_b_ref[i]
            h2 = _ln(feat, n2_s_ref[i], n2_b_ref[i])
            gm = _gelu(_dotb(h2, fc1_w_ref[i]) + fc1_b_ref[i])
            feat = feat + _dotb(gm, fc2_w_ref[i]) + fc2_b_ref[i]

        feat = _ln(feat, dn_s_ref[...], dn_b_ref[...])
        mfull = _dotb(feat, proj_m_ref[...])
        pfull = _dotb(feat, proj_p_ref[...])
        mfn = mfull * jax.lax.rsqrt(_dotb(mfull * mfull, ones48))
        pfn = pfull * jax.lax.rsqrt(_dotb(pfull * pfull, ones48))
        pf = _dot_tn(selT, pfn)
        pm = _dotb_nt(mfn, pf)
        pm = _ln(pm, mn_s_ref[...], mn_b_ref[...])
        gpm = jnp.where(rowmask, _gelu(pm), 0.0)

        r = _dotb_tn(gpm, w3f_ref[...])
        t = jnp.sum(r * amask, axis=0, keepdims=True)
        vars_row = jnp.clip(_dot(t, bmask) + mlp_b_ref[...], 0.0, 4.0)

        pts = pts_ref[g]
        var_col = _dot_nt(eye48, vars_row)

        for phase in range(2):
            p0 = phase * HALF
            pr = pts[p0:p0 + HALF, 0:1]
            pc = pts[p0:p0 + HALF, 1:2]
            vcol = var_col[p0:p0 + HALF]
            valid = jnp.where(jnp.maximum(pr, pc) > 0, 1.0, 0.0)
            cum = _dot(lower_tri, valid)
            rank = jnp.clip(cum - 1.0, 0.0, HALF - 1.0)
            onehot = jnp.where(jj == rank.astype(jnp.int32), 1.0, 0.0)
            var_p = _dot(onehot, vcol) + VAR_BIAS
            v2 = 2.0 * var_p * var_p
            nv = jnp.sum(valid)
            sc = valid / (math.pi * v2 * jnp.maximum(nv, 1.0))
            ar = jnp.exp(-((col_iota - pr) ** 2) / v2) * sc
            ac = jnp.exp(-((col_iota - pc) ** 2) / v2)
            hmap = _dot_tn(ar, ac)
            mn = jnp.min(hmap)
            mx = jnp.max(hmap)
            denom = jnp.where(mx > mn, mx - mn, 1.0)
            res = 2.0 * (hmap - mn) / denom - 1.0
            out_ref[g, phase] = jnp.where(nv > 0, res, 0.0)


def kernel(x, mask, points, conv_w, conv_b, plin_w, plin_b, n1_s, n1_b,
           qkv_w, qkv_b, ap_w, ap_b, n2_s, n2_b, fc1_w, fc1_b, fc2_w, fc2_b,
           proj_mask, proj_points, dn_s, dn_b, mn_s, mn_b, mlp_w, mlp_b):
    Bsz = x.shape[0]
    nh = IMG // PATCH
    patches = mask.reshape(Bsz, nh, PATCH, nh, PATCH).transpose(0, 1, 3, 2, 4)
    patches = patches.reshape(Bsz, NPATCH, PATCH * PATCH)
    patches = jnp.pad(patches, ((0, 0), (0, SEQP - NPATCH), (0, 0)))
    conv_wf = conv_w.reshape(D, PATCH * PATCH).T
    w3f = mlp_w.reshape(NPATCH, NPTS, NPTS).reshape(NPATCH, NPTS * NPTS)
    w3f = jnp.pad(w3f, ((0, SEQP - NPATCH), (0, 0)))

    full = lambda a: pl.BlockSpec(a.shape, lambda b: (0,) * a.ndim)
    args = [
        (patches, pl.BlockSpec((BB, SEQP, PATCH * PATCH), lambda b: (b, 0, 0))),
        (points, pl.BlockSpec((BB, NPTS, 3), lambda b: (b, 0, 0))),
        (conv_wf, full(conv_wf)),
        (conv_b.reshape(1, D), None),
        (plin_w, full(plin_w)),
        (plin_b.reshape(1, D), None),
        (n1_s.reshape(L, 1, D), None),
        (n1_b.reshape(L, 1, D), None),
        (qkv_w, full(qkv_w)),
        (qkv_b.reshape(L, 1, 3 * D), None),
        (ap_w, full(ap_w)),
        (ap_b.reshape(L, 1, D), None),
        (n2_s.reshape(L, 1, D), None),
        (n2_b.reshape(L, 1, D), None),
        (fc1_w, full(fc1_w)),
        (fc1_b.reshape(L, 1, MLPD), None),
        (fc2_w, full(fc2_w)),
        (fc2_b.reshape(L, 1, D), None),
        (proj_mask, full(proj_mask)),
        (proj_points, full(proj_points)),
        (dn_s.reshape(1, D), None),
        (dn_b.reshape(1, D), None),
        (mn_s.reshape(1, NPTS), None),
        (mn_b.reshape(1, NPTS), None),
        (w3f, full(w3f)),
        (mlp_b.reshape(1, NPTS), None),
    ]
    ins = [a for a, _ in args]
    specs = [s if s is not None else full(a) for a, s in args]
    out = pl.pallas_call(
        _fused_kernel,
        grid=(Bsz // BB,),
        in_specs=specs,
        out_specs=pl.BlockSpec((BB, 2, IMG, IMG), lambda b: (b, 0, 0, 0)),
        out_shape=jax.ShapeDtypeStruct((Bsz, 2, IMG, IMG), jnp.float32),
        compiler_params=pltpu.CompilerParams(
            dimension_semantics=('arbitrary',),
            vmem_limit_bytes=56 * 1024 * 1024,
        ),
    )(*ins)
    return out
